# baseline (speedup 1.0000x reference)
import jax
import jax.numpy as jnp
from jax import lax
from jax.experimental import pallas as pl
from jax.experimental.pallas import tpu as pltpu
from jax.experimental.pallas import tpu_sc as plsc

VOCAB = 100000
NGRAM_VOCAB = 1000000
DIM = 64
BATCH = 4096
NGRAMS = 20

NC, NS, L = 2, 16, 16
NW = NC * NS
BW = BATCH // NW
DV = DIM // L
NBUF = 3

WBLK = 512


def _tc_transpose_body(inr, outr):
    x = inr[...]
    y = x.reshape(DIM, WBLK // 2, 2)
    y = jnp.transpose(y, (1, 2, 0))
    outr[...] = y.reshape(WBLK // 2, 2 * DIM)


def _tc_transpose(net):
    return pl.pallas_call(
        _tc_transpose_body,
        grid=(pl.cdiv(NGRAM_VOCAB, WBLK),),
        in_specs=[pl.BlockSpec((DIM, WBLK), lambda i: (0, i))],
        out_specs=pl.BlockSpec((WBLK // 2, 2 * DIM), lambda i: (i, 0)),
        out_shape=jax.ShapeDtypeStruct((NGRAM_VOCAB // 2, 2 * DIM),
                                       jnp.float32),
    )(net)


def _score_body(we, ne, cwi, cnt, xwi, xnt, out,
                widx, tnv, wbuf, rows, cacc, xacc, outv,
                shacc, sem_w, gs0, gs1, gs2, ss0, ss1, ss2):
    cid = lax.axis_index("c")
    sid = lax.axis_index("s")
    wid = sid * NC + cid
    base = wid * BW
    iota = jax.lax.iota(jnp.int32, L)
    gsems = (gs0, gs1, gs2)
    ssems = (ss0, ss1, ss2)

    for k, (wsrc, nsrc) in enumerate(((cwi, cnt), (xwi, xnt))):
        pltpu.sync_copy(wsrc.at[pl.ds(base, BW)], widx.at[k])
        pltpu.sync_copy(nsrc.at[:, pl.ds(base, BW)], tnv.at[k])

    for k in range(2):
        rowbase = (sid * 2 + k) * BW
        for i in range(BW // L):
            widx[2 + k, pl.ds(i * L, L)] = rowbase + i * L + iota

    for k in range(2):
        pltpu.async_copy(we.at[widx.at[k]], wbuf, sem_w).wait()
        pltpu.sync_copy(wbuf, shacc.at[widx.at[2 + k]])

    steps = [(k, g) for g in range(NGRAMS) for k in range(2)]
    gather_cp = [None] * NBUF
    sct_cp = [None] * NBUF

    def fire_gather(step_i):
        k, g = steps[step_i]
        j = step_i % NBUF
        gather_cp[j] = pltpu.async_copy(
            ne.at[tnv.at[k, g]], rows.at[j], gsems[j])

    for i in range(NBUF):
        fire_gather(i)

    for i in range(len(steps)):
        k, g = steps[i]
        j = i % NBUF
        gather_cp[j].wait()
        sct_cp[j] = pltpu.async_copy(
            rows.at[j], shacc.at[widx.at[2 + k]], ssems[j], add=True)
        if i + NBUF < len(steps):
            sct_cp[j].wait()
            sct_cp[j] = None
            fire_gather(i + NBUF)

    for j in range(NBUF):
        if sct_cp[j] is not None:
            sct_cp[j].wait()

    pltpu.sync_copy(shacc.at[pl.ds((sid * 2) * BW, BW), :], cacc)
    pltpu.sync_copy(shacc.at[pl.ds((sid * 2 + 1) * BW, BW), :], xacc)

    for b0 in range(BW // L):
        ridx = b0 * L + iota

        @pl.loop(0, DIM, init_carry=jnp.zeros((L,), jnp.float32))
        def s(d, s):
            col = jnp.full((L,), 0, jnp.int32) + d
            c = plsc.load_gather(cacc, [ridx, col])
            x = plsc.load_gather(xacc, [ridx, col])
            return s + c * x

        outv[pl.ds(b0 * L, L)] = s

    pltpu.sync_copy(outv, out.at[pl.ds(base, BW)])


@jax.jit
def _run(we, ne_t, cwi, cnt, xwi, xnt):
    mesh = plsc.VectorSubcoreMesh(core_axis_name="c", subcore_axis_name="s",
                                  num_cores=NC, num_subcores=NS)
    ne_lin = _tc_transpose(ne_t).reshape(NGRAM_VOCAB, DIM)

    score = pl.kernel(
        _score_body,
        out_type=jax.ShapeDtypeStruct((BATCH,), jnp.float32),
        mesh=mesh,
        compiler_params=pltpu.CompilerParams(
            needs_layout_passes=False, use_tc_tiling_on_sc=False),
        scratch_types=[
            pltpu.VMEM((4, BW), jnp.int32),
            pltpu.VMEM((2, NGRAMS, BW), jnp.int32),
            pltpu.VMEM((BW, DIM), jnp.float32),
            pltpu.VMEM((NBUF, BW, DIM), jnp.float32),
            pltpu.VMEM((BW, DIM), jnp.float32),
            pltpu.VMEM((BW, DIM), jnp.float32),
            pltpu.VMEM((BW,), jnp.float32),
            pltpu.VMEM_SHARED((NS * 2 * BW, DIM), jnp.float32),
            pltpu.SemaphoreType.DMA,
            pltpu.SemaphoreType.DMA,
            pltpu.SemaphoreType.DMA,
            pltpu.SemaphoreType.DMA,
            pltpu.SemaphoreType.DMA,
            pltpu.SemaphoreType.DMA,
            pltpu.SemaphoreType.DMA,
        ],
    )
    return score(we, ne_lin, cwi, cnt, xwi, xnt)


def kernel(word_embeddings, ngram_embeddings, center_word_idx,
           center_ngram_idxs, context_word_idx, context_ngram_idxs):
    return _run(
        word_embeddings,
        ngram_embeddings.T,
        center_word_idx.astype(jnp.int32),
        center_ngram_idxs.astype(jnp.int32).T,
        context_word_idx.astype(jnp.int32),
        context_ngram_idxs.astype(jnp.int32).T)


# speedup vs baseline: 13.2755x; 13.2755x over previous
import jax
import jax.numpy as jnp
from jax import lax
from jax.experimental import pallas as pl
from jax.experimental.pallas import tpu as pltpu
from jax.experimental.pallas import tpu_sc as plsc

VOCAB = 100000
NGRAM_VOCAB = 1000000
DIM = 64
BATCH = 4096
NGRAMS = 20

NC, NS, L = 2, 16, 16
NW = NC * NS
BW = BATCH // NW
DV = DIM // L
NBUF = 3

def _score_body(we, ne, cwi, cnt, xwi, xnt, out,
                widx, tnv, wbuf, rows, cacc, xacc, outv,
                shacc, sem_w, gs0, gs1, gs2, ss0, ss1, ss2):
    cid = lax.axis_index("c")
    sid = lax.axis_index("s")
    wid = sid * NC + cid
    base = wid * BW
    iota = jax.lax.iota(jnp.int32, L)
    gsems = (gs0, gs1, gs2)
    ssems = (ss0, ss1, ss2)

    for k, (wsrc, nsrc) in enumerate(((cwi, cnt), (xwi, xnt))):
        pltpu.sync_copy(wsrc.at[pl.ds(base, BW)], widx.at[k])
        pltpu.sync_copy(nsrc.at[:, pl.ds(base, BW)], tnv.at[k])

    for k in range(2):
        rowbase = (sid * 2 + k) * BW
        for i in range(BW // L):
            widx[2 + k, pl.ds(i * L, L)] = rowbase + i * L + iota

    for k in range(2):
        pltpu.async_copy(we.at[widx.at[k]], wbuf, sem_w).wait()
        pltpu.sync_copy(wbuf, shacc.at[widx.at[2 + k]])

    steps = [(k, g) for g in range(NGRAMS) for k in range(2)]
    gather_cp = [None] * NBUF
    sct_cp = [None] * NBUF

    def fire_gather(step_i):
        k, g = steps[step_i]
        j = step_i % NBUF
        gather_cp[j] = pltpu.async_copy(
            ne.at[tnv.at[k, g]], rows.at[j], gsems[j])

    for i in range(NBUF):
        fire_gather(i)

    for i in range(len(steps)):
        k, g = steps[i]
        j = i % NBUF
        gather_cp[j].wait()
        sct_cp[j] = pltpu.async_copy(
            rows.at[j], shacc.at[widx.at[2 + k]], ssems[j], add=True)
        if i + NBUF < len(steps):
            sct_cp[j].wait()
            sct_cp[j] = None
            fire_gather(i + NBUF)

    for j in range(NBUF):
        if sct_cp[j] is not None:
            sct_cp[j].wait()

    pltpu.sync_copy(shacc.at[pl.ds((sid * 2) * BW, BW), :], cacc)
    pltpu.sync_copy(shacc.at[pl.ds((sid * 2 + 1) * BW, BW), :], xacc)

    for b0 in range(BW // L):
        ridx = b0 * L + iota

        @pl.loop(0, DIM, init_carry=jnp.zeros((L,), jnp.float32))
        def s(d, s):
            col = jnp.full((L,), 0, jnp.int32) + d
            c = plsc.load_gather(cacc, [ridx, col])
            x = plsc.load_gather(xacc, [ridx, col])
            return s + c * x

        outv[pl.ds(b0 * L, L)] = s

    pltpu.sync_copy(outv, out.at[pl.ds(base, BW)])


@jax.jit
def _run(we, ne, cwi, cnt, xwi, xnt):
    mesh = plsc.VectorSubcoreMesh(core_axis_name="c", subcore_axis_name="s",
                                  num_cores=NC, num_subcores=NS)
    score = pl.kernel(
        _score_body,
        out_type=jax.ShapeDtypeStruct((BATCH,), jnp.float32),
        mesh=mesh,
        compiler_params=pltpu.CompilerParams(
            needs_layout_passes=False, use_tc_tiling_on_sc=False),
        scratch_types=[
            pltpu.VMEM((4, BW), jnp.int32),
            pltpu.VMEM((2, NGRAMS, BW), jnp.int32),
            pltpu.VMEM((BW, DIM), jnp.float32),
            pltpu.VMEM((NBUF, BW, DIM), jnp.float32),
            pltpu.VMEM((BW, DIM), jnp.float32),
            pltpu.VMEM((BW, DIM), jnp.float32),
            pltpu.VMEM((BW,), jnp.float32),
            pltpu.VMEM_SHARED((NS * 2 * BW, DIM), jnp.float32),
            pltpu.SemaphoreType.DMA,
            pltpu.SemaphoreType.DMA,
            pltpu.SemaphoreType.DMA,
            pltpu.SemaphoreType.DMA,
            pltpu.SemaphoreType.DMA,
            pltpu.SemaphoreType.DMA,
            pltpu.SemaphoreType.DMA,
        ],
    )
    return score(we, ne, cwi, cnt, xwi, xnt)


def kernel(word_embeddings, ngram_embeddings, center_word_idx,
           center_ngram_idxs, context_word_idx, context_ngram_idxs):
    return _run(
        word_embeddings,
        ngram_embeddings,
        center_word_idx.astype(jnp.int32),
        center_ngram_idxs.astype(jnp.int32).T,
        context_word_idx.astype(jnp.int32),
        context_ngram_idxs.astype(jnp.int32).T)
